# Initial kernel scaffold; baseline (speedup 1.0000x reference)
import jax
import jax.numpy as jnp
from jax.experimental import pallas as pl


def kernel(x):
    raise NotImplementedError("write your pallas kernel here")


# speedup vs baseline: 148.6539x; 148.6539x over previous
import functools

import jax
import jax.numpy as jnp
from jax import lax
from jax.experimental import pallas as pl
from jax.experimental.pallas import tpu as pltpu
from jax.experimental.pallas import tpu_sc as plsc

N = 16777216
BINS = 2048
NC = 2
NS = 16
L = 16
NW = NC * NS
PER_W = N // NW
CHUNK = 32768
NCHUNK = PER_W // CHUNK
VEC_PER_CHUNK = CHUNK // L
STRIPE = BINS // NS

_mesh = plsc.VectorSubcoreMesh(
    core_axis_name="c", subcore_axis_name="s", num_cores=NC, num_subcores=NS
)
_params = pltpu.CompilerParams(needs_layout_passes=False)


GP = 16
TCROWS = N // 128
RPB = TCROWS // GP


@functools.partial(
    pl.pallas_call,
    grid=(GP,),
    in_specs=[pl.BlockSpec((RPB, 128), lambda i: (i, 0))],
    out_specs=pl.BlockSpec((2, 128), lambda i: (0, 0)),
    out_shape=jax.ShapeDtypeStruct((2, 128), jnp.float32),
    scratch_shapes=[pltpu.SMEM((2,), jnp.float32)],
)
def _minmax_tc(x_ref, o_ref, acc):
    i = pl.program_id(0)
    blk = x_ref[...]
    bmin = jnp.min(blk)
    bmax = jnp.max(blk)

    @pl.when(i == 0)
    def _():
        acc[0] = bmin
        acc[1] = bmax

    @pl.when(i > 0)
    def _():
        acc[0] = jnp.minimum(acc[0], bmin)
        acc[1] = jnp.maximum(acc[1], bmax)

    @pl.when(i == GP - 1)
    def _():
        o_ref[0, :] = jnp.full((128,), acc[0], jnp.float32)
        o_ref[1, :] = jnp.full((128,), acc[1], jnp.float32)


@functools.partial(
    pl.kernel,
    out_type=jax.ShapeDtypeStruct((NC, BINS), jnp.float32),
    mesh=_mesh,
    compiler_params=_params,
    scratch_types=[
        pltpu.VMEM((CHUNK,), jnp.float32),
        pltpu.VMEM((CHUNK,), jnp.float32),
        pltpu.VMEM((2, L), jnp.float32),
        pltpu.VMEM((L * BINS,), jnp.float32),
        pltpu.VMEM((BINS + L,), jnp.float32),
        pltpu.VMEM((STRIPE,), jnp.float32),
        pltpu.VMEM((STRIPE,), jnp.float32),
        pltpu.VMEM_SHARED((NS, BINS), jnp.float32),
        pltpu.SemaphoreType.DMA,
        pltpu.SemaphoreType.DMA,
    ],
)
def _hist(
    x_hbm,
    mm_hbm,
    hist_out_hbm,
    buf0,
    buf1,
    part_v,
    hist16_v,
    hsum_v,
    acc_v,
    tmp_v,
    shared,
    sem0,
    sem1,
):
    cid = lax.axis_index("c")
    sid = lax.axis_index("s")
    wid = sid * NC + cid
    base = pl.multiple_of(wid * PER_W, CHUNK)
    bufs = (buf0, buf1)
    sems = (sem0, sem1)

    copies = [None, None]
    copies[0] = pltpu.async_copy(x_hbm.at[pl.ds(base, CHUNK)], buf0, sem0)

    pltpu.sync_copy(mm_hbm.at[0, pl.ds(0, L)], part_v.at[0])
    pltpu.sync_copy(mm_hbm.at[1, pl.ds(0, L)], part_v.at[1])
    gmin = part_v[0, :][0]
    gmax = part_v[1, :][0]
    bw = (gmax - gmin) * jnp.float32(1.0 / BINS)
    safe_bw = jnp.where(bw <= 0.0, jnp.float32(1.0), bw)

    zeros = jnp.zeros((L,), jnp.float32)

    def zbody(i, _):
        hist16_v[pl.ds(i * L, L)] = zeros
        return 0

    lax.fori_loop(0, (L * BINS) // L, zbody, 0)

    ones = jnp.ones((L,), jnp.float32)
    lane_iota = lax.iota(jnp.int32, L)
    max_binf = jnp.full((L,), float(BINS - 1), jnp.float32)

    U = 8
    UNROLL = 4
    for c in range(NCHUNK):
        cur = c % 2
        copies[cur].wait()
        if c + 1 < NCHUNK:
            nxt = (c + 1) % 2
            copies[nxt] = pltpu.async_copy(
                x_hbm.at[pl.ds(base + (c + 1) * CHUNK, CHUNK)], bufs[nxt], sems[nxt]
            )
        buf = bufs[cur]

        @plsc.parallel_loop(0, VEC_PER_CHUNK // U, unroll=UNROLL)
        def _(i, buf=buf):
            b = i * (L * U)
            vs = [buf[pl.ds(b + u * L, L)] for u in range(U)]
            idxs = []
            for v in vs:
                t = (v - gmin) / safe_bw
                t = jnp.minimum(t, max_binf)
                idxs.append((t.astype(jnp.int32) << 4) | lane_iota)
            for idx in idxs:
                plsc.addupdate_scatter(hist16_v, [idx], ones)

    last_lane = lane_iota == (L - 1)

    def fbody(b, _):
        w = hist16_v[pl.ds(b * L, L)]
        cs = plsc.cumsum(w)
        plsc.store_compressed(hsum_v.at[pl.ds(b, L)], cs, mask=last_lane)
        return 0

    lax.fori_loop(0, BINS, fbody, 0)

    pltpu.sync_copy(hsum_v.at[pl.ds(0, BINS)], shared.at[sid])
    plsc.subcore_barrier()

    col = pl.multiple_of(sid * STRIPE, STRIPE)
    pltpu.sync_copy(shared.at[0, pl.ds(col, STRIPE)], acc_v)
    for w in range(1, NS):
        pltpu.sync_copy(shared.at[w, pl.ds(col, STRIPE)], tmp_v)
        for j in range(STRIPE // L):
            acc_v[pl.ds(j * L, L)] = acc_v[pl.ds(j * L, L)] + tmp_v[pl.ds(j * L, L)]
    pltpu.sync_copy(acc_v, hist_out_hbm.at[cid, pl.ds(col, STRIPE)])


@jax.jit
def kernel(x):
    mm_tc = _minmax_tc(x.reshape(TCROWS, 128))
    hist2 = _hist(x, mm_tc)
    histogram = hist2[0] + hist2[1]
    return histogram, mm_tc[0, 0], mm_tc[1, 0]
